# baseline (speedup 1.0000x reference)
import jax
import jax.numpy as jnp
from jax.experimental import pallas as pl
from jax.experimental.pallas import tpu as pltpu

N = 65536
C = 256
D = 64
K_PROJ = 5
CELL = 5.0
N_ROUNDS = 30
NB = 8192
BN = 512
NI = N // BN
GR = 6
NG = N_ROUNDS // GR


def _make_consts():
    primes = jnp.zeros((1, 128), jnp.int32).at[0, :K_PROJ].set(
        jnp.array([73856093, 19349663, 83492791, 49979687, 86028121], jnp.int32)
    )
    base = jax.random.key(42)
    Rs, offs = [], []
    for r in range(N_ROUNDS):
        kr = jax.random.fold_in(base, r)
        ka, kb = jax.random.split(kr)
        R = jax.random.normal(ka, (D, K_PROJ), jnp.float32) / jnp.sqrt(
            jnp.asarray(D, jnp.float32)
        )
        off = jax.random.uniform(kb, (K_PROJ,), jnp.float32) * CELL
        Rs.append(jnp.zeros((D, 128), jnp.float32).at[:, :K_PROJ].set(R))
        offs.append(jnp.zeros((1, 128), jnp.float32).at[0, :K_PROJ].set(off))
    return jnp.stack(Rs), jnp.stack(offs), primes


def _hash(ref_blk, R, off, primes):
    proj = jnp.dot(ref_blk, R, preferred_element_type=jnp.float32) + off
    cells = jnp.floor(proj / CELL).astype(jnp.int32)
    return jnp.sum(cells * primes, axis=1) & (NB - 1)


def _seg_body(ref_ref, u_ref, R_ref, o_ref, p_ref, means_ref, sums, cnts):
    i = pl.program_id(1)
    h = _hash(ref_ref[...], R_ref[0], o_ref[0], p_ref[...])
    iota = jax.lax.broadcasted_iota(jnp.int32, (NB, BN), 0)
    oh = (iota == h[None, :]).astype(jnp.bfloat16)
    contrib = jnp.dot(oh, u_ref[...].astype(jnp.bfloat16),
                      preferred_element_type=jnp.float32)
    csum = jnp.dot(oh, jnp.ones((BN, 8), jnp.bfloat16),
                   preferred_element_type=jnp.float32)

    @pl.when(i == 0)
    def _():
        sums[...] = contrib
        cnts[...] = csum

    @pl.when(i != 0)
    def _():
        sums[...] += contrib
        cnts[...] += csum

    @pl.when(i == NI - 1)
    def _():
        means_ref[0] = (
            sums[...] / jnp.maximum(cnts[...][:, :1], 1.0)
        ).astype(jnp.bfloat16)


def _gather_body(acc_ref, ref_ref, R_ref, o_ref, p_ref, means_ref, out_ref):
    total = acc_ref[...]
    for rr in range(GR):
        h = _hash(ref_ref[...], R_ref[rr], o_ref[rr], p_ref[...])
        iota = jax.lax.broadcasted_iota(jnp.int32, (BN, NB), 1)
        oh = (iota == h[:, None]).astype(jnp.bfloat16)
        total += jnp.dot(oh, means_ref[rr],
                         preferred_element_type=jnp.float32)
    out_ref[...] = total


def kernel(U, ref):
    R_all, offs_all, primes = _make_consts()

    means = pl.pallas_call(
        _seg_body,
        grid=(N_ROUNDS, NI),
        in_specs=[
            pl.BlockSpec((BN, D), lambda r, i: (i, 0)),
            pl.BlockSpec((BN, C), lambda r, i: (i, 0)),
            pl.BlockSpec((1, D, 128), lambda r, i: (r, 0, 0)),
            pl.BlockSpec((1, 1, 128), lambda r, i: (r, 0, 0)),
            pl.BlockSpec((1, 128), lambda r, i: (0, 0)),
        ],
        out_specs=pl.BlockSpec((1, NB, C), lambda r, i: (r, 0, 0)),
        out_shape=jax.ShapeDtypeStruct((N_ROUNDS, NB, C), jnp.bfloat16),
        scratch_shapes=[
            pltpu.VMEM((NB, C), jnp.float32),
            pltpu.VMEM((NB, 8), jnp.float32),
        ],
    )(ref, U, R_all, offs_all, primes)

    acc = jnp.zeros((N, C), jnp.float32)
    for g in range(NG):
        s = slice(g * GR, (g + 1) * GR)
        acc = pl.pallas_call(
            _gather_body,
            grid=(NI,),
            in_specs=[
                pl.BlockSpec((BN, C), lambda i: (i, 0)),
                pl.BlockSpec((BN, D), lambda i: (i, 0)),
                pl.BlockSpec((GR, D, 128), lambda i: (0, 0, 0)),
                pl.BlockSpec((GR, 1, 128), lambda i: (0, 0, 0)),
                pl.BlockSpec((1, 128), lambda i: (0, 0)),
                pl.BlockSpec((GR, NB, C), lambda i: (0, 0, 0)),
            ],
            out_specs=pl.BlockSpec((BN, C), lambda i: (i, 0)),
            out_shape=jax.ShapeDtypeStruct((N, C), jnp.float32),
            input_output_aliases={0: 0},
        )(acc, ref, R_all[s], offs_all[s], primes, means[s])

    return acc / jnp.asarray(N_ROUNDS, jnp.float32) - U


# speedup vs baseline: 1.1047x; 1.1047x over previous
import jax
import jax.numpy as jnp
from jax.experimental import pallas as pl
from jax.experimental.pallas import tpu as pltpu

N = 65536
C = 256
D = 64
K_PROJ = 5
CELL = 5.0
N_ROUNDS = 30
NB = 8192
BN = 512
NI = N // BN
GR = 6
NG = N_ROUNDS // GR


def _make_consts():
    primes = jnp.zeros((1, 128), jnp.int32).at[0, :K_PROJ].set(
        jnp.array([73856093, 19349663, 83492791, 49979687, 86028121], jnp.int32)
    )
    base = jax.random.key(42)
    Rs, offs = [], []
    for r in range(N_ROUNDS):
        kr = jax.random.fold_in(base, r)
        ka, kb = jax.random.split(kr)
        R = jax.random.normal(ka, (D, K_PROJ), jnp.float32) / jnp.sqrt(
            jnp.asarray(D, jnp.float32)
        )
        off = jax.random.uniform(kb, (K_PROJ,), jnp.float32) * CELL
        Rs.append(jnp.zeros((D, 128), jnp.float32).at[:, :K_PROJ].set(R))
        offs.append(jnp.zeros((1, 128), jnp.float32).at[0, :K_PROJ].set(off))
    return jnp.stack(Rs), jnp.stack(offs), primes


def _hash(ref_blk, R, off, primes):
    proj = jnp.dot(ref_blk, R, preferred_element_type=jnp.float32) + off
    cells = jnp.floor(proj / CELL).astype(jnp.int32)
    return jnp.sum(cells * primes, axis=1) & (NB - 1)


def _seg_body(ref_ref, u_ref, R_ref, o_ref, p_ref, means_ref, sums, cnts):
    i = pl.program_id(1)
    h = _hash(ref_ref[...], R_ref[0], o_ref[0], p_ref[...])
    iota = jax.lax.broadcasted_iota(jnp.int32, (NB, BN), 0)
    oh = (iota == h[None, :]).astype(jnp.bfloat16)
    contrib = jnp.dot(oh, u_ref[...],
                      preferred_element_type=jnp.float32)
    csum = jnp.sum(oh.astype(jnp.float32), axis=1, keepdims=True) + jnp.zeros(
        (NB, 8), jnp.float32
    )

    @pl.when(i == 0)
    def _():
        sums[...] = contrib
        cnts[...] = csum

    @pl.when(i != 0)
    def _():
        sums[...] += contrib
        cnts[...] += csum

    @pl.when(i == NI - 1)
    def _():
        means_ref[0] = (
            sums[...] / jnp.maximum(cnts[...][:, :1], 1.0)
        ).astype(jnp.bfloat16)


def _gather_body(acc_ref, ref_ref, R_ref, o_ref, p_ref, means_ref, out_ref):
    total = acc_ref[...]
    for rr in range(GR):
        h = _hash(ref_ref[...], R_ref[rr], o_ref[rr], p_ref[...])
        iota = jax.lax.broadcasted_iota(jnp.int32, (BN, NB), 1)
        oh = (iota == h[:, None]).astype(jnp.bfloat16)
        total += jnp.dot(oh, means_ref[rr],
                         preferred_element_type=jnp.float32)
    out_ref[...] = total


def kernel(U, ref):
    R_all, offs_all, primes = _make_consts()

    means = pl.pallas_call(
        _seg_body,
        grid=(N_ROUNDS, NI),
        in_specs=[
            pl.BlockSpec((BN, D), lambda r, i: (i, 0)),
            pl.BlockSpec((BN, C), lambda r, i: (i, 0)),
            pl.BlockSpec((1, D, 128), lambda r, i: (r, 0, 0)),
            pl.BlockSpec((1, 1, 128), lambda r, i: (r, 0, 0)),
            pl.BlockSpec((1, 128), lambda r, i: (0, 0)),
        ],
        out_specs=pl.BlockSpec((1, NB, C), lambda r, i: (r, 0, 0)),
        out_shape=jax.ShapeDtypeStruct((N_ROUNDS, NB, C), jnp.bfloat16),
        scratch_shapes=[
            pltpu.VMEM((NB, C), jnp.float32),
            pltpu.VMEM((NB, 8), jnp.float32),
        ],
    )(ref, U.astype(jnp.bfloat16), R_all, offs_all, primes)

    acc = jnp.zeros((N, C), jnp.float32)
    for g in range(NG):
        s = slice(g * GR, (g + 1) * GR)
        acc = pl.pallas_call(
            _gather_body,
            grid=(NI,),
            in_specs=[
                pl.BlockSpec((BN, C), lambda i: (i, 0)),
                pl.BlockSpec((BN, D), lambda i: (i, 0)),
                pl.BlockSpec((GR, D, 128), lambda i: (0, 0, 0)),
                pl.BlockSpec((GR, 1, 128), lambda i: (0, 0, 0)),
                pl.BlockSpec((1, 128), lambda i: (0, 0)),
                pl.BlockSpec((GR, NB, C), lambda i: (0, 0, 0)),
            ],
            out_specs=pl.BlockSpec((BN, C), lambda i: (i, 0)),
            out_shape=jax.ShapeDtypeStruct((N, C), jnp.float32),
            input_output_aliases={0: 0},
        )(acc, ref, R_all[s], offs_all[s], primes, means[s])

    return acc / jnp.asarray(N_ROUNDS, jnp.float32) - U
